# Initial kernel scaffold; baseline (speedup 1.0000x reference)
import jax
import jax.numpy as jnp
from jax.experimental import pallas as pl


def kernel(h, e, weight, attention, edge_index, rel):
    raise NotImplementedError("write your pallas kernel here")


# speedup vs baseline: 4.5833x; 4.5833x over previous
import functools

import jax
import jax.numpy as jnp
from jax import lax
from jax.experimental import pallas as pl
from jax.experimental.pallas import tpu as pltpu
from jax.experimental.pallas import tpu_sc as plsc

N_NODES = 10000
N_EDGES = 320000
D = 128
NUM_RELS = 100

NC = 2
NS = 16
NW = NC * NS
EPW = N_EDGES // NW
C = 80
K = EPW // C
G = 5
B = K // G
ROWS_PER_SUB = 624
TAIL_ROWS = N_NODES - NS * ROWS_PER_SUB
TAIL_BASE = NS * ROWS_PER_SUB


def _sc_agg(h, e, weight, attention, src_r, dst_r, rel_r):
    mesh = plsc.VectorSubcoreMesh(core_axis_name="c", subcore_axis_name="s")

    @functools.partial(
        pl.kernel,
        out_type=jax.ShapeDtypeStruct((NC, N_NODES, D), jnp.float32),
        mesh=mesh,
        scratch_types=[
            pltpu.VMEM_SHARED((N_NODES, D), jnp.float32),
            pltpu.VMEM((G, C), jnp.int32),
            pltpu.VMEM((G, C), jnp.int32),
            pltpu.VMEM((G, C), jnp.int32),
            pltpu.VMEM((C, D), jnp.float32),
            pltpu.VMEM((C, D), jnp.float32),
            pltpu.VMEM((C, D), jnp.float32),
            pltpu.VMEM((C, D), jnp.float32),
            pltpu.SemaphoreType.DMA,
            pltpu.SemaphoreType.DMA,
            pltpu.SemaphoreType.DMA,
            pltpu.SemaphoreType.DMA,
        ],
    )
    def k(h_hbm, e_hbm, w_hbm, a_hbm, src_hbm, dst_hbm, rel_hbm, parts_hbm,
          agg, src_v, dst_v, rel_v, node_v, e_v, w_v, a_v,
          sem0, sem1, sem2, sem3):
        c = lax.axis_index("c")
        s = lax.axis_index("s")
        wid = c * NS + s

        row0 = s * ROWS_PER_SUB
        pltpu.sync_copy(h_hbm.at[pl.ds(row0, ROWS_PER_SUB)],
                        agg.at[pl.ds(row0, ROWS_PER_SUB)])

        @pl.when(s == 0)
        def _():
            pltpu.sync_copy(h_hbm.at[pl.ds(TAIL_BASE, TAIL_ROWS)],
                            agg.at[pl.ds(TAIL_BASE, TAIL_ROWS)])

        plsc.subcore_barrier()

        @pl.loop(0, B)
        def _(b):
            pltpu.sync_copy(src_hbm.at[wid, b], src_v)
            pltpu.sync_copy(dst_hbm.at[wid, b], dst_v)
            pltpu.sync_copy(rel_hbm.at[wid, b], rel_v)

            @pl.loop(0, G)
            def _(g):
                ebase = wid * EPW + (b * G + g) * C
                d0 = pltpu.async_copy(h_hbm.at[src_v.at[g]], node_v, sem0)
                d1 = pltpu.async_copy(w_hbm.at[rel_v.at[g]], w_v, sem1)
                d2 = pltpu.async_copy(a_hbm.at[rel_v.at[g]], a_v, sem2)
                d3 = pltpu.async_copy(e_hbm.at[pl.ds(ebase, C)], e_v, sem3)
                d0.wait()
                d1.wait()
                d2.wait()
                d3.wait()

                @pl.loop(0, C)
                def _(r):
                    for dd in range(D // 16):
                        sl = pl.ds(dd * 16, 16)
                        node_v[r, sl] = (node_v[r, sl] * w_v[r, sl]
                                         + e_v[r, sl] * a_v[r, sl])

                pltpu.sync_copy(node_v, agg.at[dst_v.at[g]], add=True)

        plsc.subcore_barrier()
        pltpu.sync_copy(agg.at[pl.ds(row0, ROWS_PER_SUB)],
                        parts_hbm.at[c, pl.ds(row0, ROWS_PER_SUB)])

        @pl.when(s == 0)
        def _():
            pltpu.sync_copy(agg.at[pl.ds(TAIL_BASE, TAIL_ROWS)],
                            parts_hbm.at[c, pl.ds(TAIL_BASE, TAIL_ROWS)])

    return k(h, e, weight, attention, src_r, dst_r, rel_r)


def _combine_kernel(parts_ref, h_ref, o_ref):
    o_ref[...] = parts_ref[0] + parts_ref[1] - h_ref[...]


def _combine(parts, h):
    bn = 2000
    return pl.pallas_call(
        _combine_kernel,
        grid=(N_NODES // bn,),
        in_specs=[
            pl.BlockSpec((NC, bn, D), lambda i: (0, i, 0)),
            pl.BlockSpec((bn, D), lambda i: (i, 0)),
        ],
        out_specs=pl.BlockSpec((bn, D), lambda i: (i, 0)),
        out_shape=jax.ShapeDtypeStruct((N_NODES, D), jnp.float32),
    )(parts, h)


def kernel(h, e, weight, attention, edge_index, rel):
    src = edge_index[0].astype(jnp.int32).reshape(NW, B, G, C)
    dst = edge_index[1].astype(jnp.int32).reshape(NW, B, G, C)
    rel32 = rel.astype(jnp.int32).reshape(NW, B, G, C)
    parts = _sc_agg(h, e, weight, attention, src, dst, rel32)
    return _combine(parts, h)
